# baseline (speedup 1.0000x reference)
import functools

import jax
import jax.numpy as jnp
from jax import lax
from jax.experimental import pallas as pl
from jax.experimental.pallas import tpu as pltpu
from jax.experimental.pallas import tpu_sc as plsc

_NUM_CORES = 2
_NUM_SUBCORES = 16
_NUM_WORKERS = _NUM_CORES * _NUM_SUBCORES
_LANES = 16


def kernel(inputs, token_table, pos_table):
    B, S = inputs.shape
    V, D = token_table.shape
    idx = inputs.reshape(-1).astype(jnp.int32)
    seq_per_w = B // _NUM_WORKERS

    mesh = plsc.VectorSubcoreMesh(core_axis_name="c", subcore_axis_name="s")

    @functools.partial(
        pl.kernel,
        mesh=mesh,
        out_type=jax.ShapeDtypeStruct((B * S, D), jnp.float32),
        scratch_types=[
            pltpu.VMEM((S,), jnp.int32),
            pltpu.VMEM((S, D), jnp.float32),
            pltpu.VMEM((S, D), jnp.float32),
            pltpu.SemaphoreType.DMA,
        ],
        compiler_params=pltpu.CompilerParams(use_tc_tiling_on_sc=False),
    )
    def sc_kernel(idx_hbm, tok_hbm, pos_hbm, out_hbm, idx_v, rows_v, pos_v, sem):
        wid = lax.axis_index("s") * _NUM_CORES + lax.axis_index("c")
        pltpu.sync_copy(pos_hbm, pos_v)

        def per_seq(s, _):
            base = (wid * seq_per_w + s) * S
            pltpu.sync_copy(idx_hbm.at[pl.ds(base, S)], idx_v)
            pltpu.async_copy(tok_hbm.at[idx_v], rows_v, sem).wait()

            def per_row(r, _):
                for c in range(D // _LANES):
                    sl = pl.ds(c * _LANES, _LANES)
                    rows_v[r, sl] = rows_v[r, sl] + pos_v[r, sl]
                return ()

            lax.fori_loop(0, S, per_row, ())
            pltpu.sync_copy(rows_v, out_hbm.at[pl.ds(base, S)])
            return ()

        lax.fori_loop(0, seq_per_w, per_seq, ())

    out = sc_kernel(idx, token_table, pos_table)
    return out.reshape(B, S, D)


# speedup vs baseline: 1.0720x; 1.0720x over previous
import functools

import jax
import jax.numpy as jnp
from jax import lax
from jax.experimental import pallas as pl
from jax.experimental.pallas import tpu as pltpu
from jax.experimental.pallas import tpu_sc as plsc

_NUM_CORES = 2
_NUM_SUBCORES = 16
_NUM_WORKERS = _NUM_CORES * _NUM_SUBCORES
_LANES = 16


def kernel(inputs, token_table, pos_table):
    B, S = inputs.shape
    V, D = token_table.shape
    idx = inputs.reshape(-1).astype(jnp.int32)
    seq_per_w = B // _NUM_WORKERS

    mesh = plsc.VectorSubcoreMesh(core_axis_name="c", subcore_axis_name="s")
    NB = 3

    @functools.partial(
        pl.kernel,
        mesh=mesh,
        out_type=jax.ShapeDtypeStruct((B * S, D), jnp.float32),
        scratch_types=[
            pltpu.VMEM((seq_per_w * S,), jnp.int32),
            [pltpu.VMEM((S, D), jnp.float32) for _ in range(NB)],
            pltpu.VMEM((S, D), jnp.float32),
            pltpu.SemaphoreType.DMA((NB,)),
            pltpu.SemaphoreType.DMA((NB,)),
        ],
        compiler_params=pltpu.CompilerParams(use_tc_tiling_on_sc=False),
    )
    def sc_kernel(idx_hbm, tok_hbm, pos_hbm, out_hbm, idx_v, rows, pos_v,
                  gsem, ssem):
        wid = lax.axis_index("s") * _NUM_CORES + lax.axis_index("c")
        base = wid * seq_per_w * S
        pltpu.sync_copy(pos_hbm, pos_v)
        pltpu.sync_copy(idx_hbm.at[pl.ds(base, seq_per_w * S)], idx_v)

        def gather_start(t):
            return pltpu.async_copy(
                tok_hbm.at[idx_v.at[pl.ds(t * S, S)]], rows[t % NB],
                gsem.at[t % NB])

        def store_start(t):
            return pltpu.async_copy(
                rows[t % NB], out_hbm.at[pl.ds(base + t * S, S)],
                ssem.at[t % NB])

        def adds(b):
            def per_row(r, _):
                for c in range(D // _LANES):
                    sl = pl.ds(c * _LANES, _LANES)
                    rows[b][r, sl] = rows[b][r, sl] + pos_v[r, sl]
                return ()

            lax.fori_loop(0, S, per_row, ())

        g = [None] * seq_per_w
        st = [None] * seq_per_w
        for t in range(min(NB, seq_per_w)):
            g[t] = gather_start(t)
        for s in range(seq_per_w):
            g[s].wait()
            adds(s % NB)
            if s >= 1:
                st[s - 1].wait()
                t = s - 1 + NB
                if t < seq_per_w:
                    g[t] = gather_start(t)
            st[s] = store_start(s)
        st[seq_per_w - 1].wait()

    out = sc_kernel(idx, token_table, pos_table)
    return out.reshape(B, S, D)
